# initial kernel scaffold (unmeasured)
import jax
import jax.numpy as jnp
from jax import lax
from jax.experimental import pallas as pl
from jax.experimental.pallas import tpu as pltpu

N_DEV = 32


def kernel(x, w_mat):
    m_per, k = x.shape
    n = w_mat.shape[1]
    n_per = n // N_DEV
    m_total = m_per * N_DEV

    def body(x_ref, w_ref, out_ref, y_ref, send_sem, recv_sem):
        my = lax.axis_index("i")

        y = jnp.dot(x_ref[...], w_ref[...], preferred_element_type=jnp.float32)
        y_ref[...] = y * jax.nn.sigmoid(y)

        barrier = pltpu.get_barrier_semaphore()
        for p in range(N_DEV):
            pl.semaphore_signal(
                barrier, inc=1,
                device_id=(p,), device_id_type=pl.DeviceIdType.MESH,
            )
        pl.semaphore_wait(barrier, N_DEV)

        for p in range(N_DEV):
            @pl.when(my != p)
            def _():
                rdma = pltpu.make_async_remote_copy(
                    src_ref=y_ref.at[:, pl.ds(p * n_per, n_per)],
                    dst_ref=out_ref.at[pl.ds(my * m_per, m_per), :],
                    send_sem=send_sem,
                    recv_sem=recv_sem,
                    device_id=(p,),
                    device_id_type=pl.DeviceIdType.MESH,
                )
                rdma.start()

        out_ref[pl.ds(my * m_per, m_per), :] = y_ref[:, pl.ds(my * n_per, n_per)]

        for _ in range(N_DEV - 1):
            wait = pltpu.make_async_remote_copy(
                src_ref=y_ref.at[:, pl.ds(0, n_per)],
                dst_ref=out_ref.at[pl.ds(0, m_per), :],
                send_sem=send_sem,
                recv_sem=recv_sem,
                device_id=(0,),
                device_id_type=pl.DeviceIdType.MESH,
            )
            wait.wait_send()
            wait.wait_recv()

    return pl.pallas_call(
        body,
        out_shape=jax.ShapeDtypeStruct((m_total, n_per), jnp.float32),
        in_specs=[
            pl.BlockSpec(memory_space=pltpu.VMEM),
            pl.BlockSpec(memory_space=pltpu.VMEM),
        ],
        out_specs=pl.BlockSpec(memory_space=pltpu.VMEM),
        scratch_shapes=[
            pltpu.VMEM((m_per, n), jnp.float32),
            pltpu.SemaphoreType.DMA,
            pltpu.SemaphoreType.DMA,
        ],
        compiler_params=pltpu.CompilerParams(collective_id=0),
    )(x, w_mat)


# baseline (device time: 31476 ns/iter reference)
import jax
import jax.numpy as jnp
from jax import lax
from jax.experimental import pallas as pl
from jax.experimental.pallas import tpu as pltpu

N_DEV = 32


def kernel(x, w_mat):
    m_per, k = x.shape
    n = w_mat.shape[1]
    n_per = n // N_DEV
    m_total = m_per * N_DEV

    def body(x_ref, w_ref, out_ref, y_ref, send_sem, recv_sem):
        my = lax.axis_index("i")

        y = jnp.dot(x_ref[...], w_ref[...], preferred_element_type=jnp.float32)
        y = y * jax.nn.sigmoid(y)
        y_ref[...] = y.reshape(m_per, N_DEV, n_per).transpose(1, 0, 2)

        barrier = pltpu.get_barrier_semaphore()
        for p in range(N_DEV):
            pl.semaphore_signal(
                barrier, inc=1,
                device_id=(p,), device_id_type=pl.DeviceIdType.MESH,
            )
        pl.semaphore_wait(barrier, N_DEV)

        for p in range(N_DEV):
            @pl.when(my != p)
            def _():
                rdma = pltpu.make_async_remote_copy(
                    src_ref=y_ref.at[p],
                    dst_ref=out_ref.at[pl.ds(my * m_per, m_per), :],
                    send_sem=send_sem,
                    recv_sem=recv_sem,
                    device_id=(p,),
                    device_id_type=pl.DeviceIdType.MESH,
                )
                rdma.start()

        out_ref[pl.ds(my * m_per, m_per), :] = y_ref[my]

        for _ in range(N_DEV - 1):
            wait = pltpu.make_async_remote_copy(
                src_ref=y_ref.at[0],
                dst_ref=out_ref.at[pl.ds(0, m_per), :],
                send_sem=send_sem,
                recv_sem=recv_sem,
                device_id=(0,),
                device_id_type=pl.DeviceIdType.MESH,
            )
            wait.wait_send()
            wait.wait_recv()

    return pl.pallas_call(
        body,
        out_shape=jax.ShapeDtypeStruct((m_total, n_per), jnp.float32),
        in_specs=[
            pl.BlockSpec(memory_space=pltpu.VMEM),
            pl.BlockSpec(memory_space=pltpu.VMEM),
        ],
        out_specs=pl.BlockSpec(memory_space=pltpu.VMEM),
        scratch_shapes=[
            pltpu.VMEM((N_DEV, m_per, n_per), jnp.float32),
            pltpu.SemaphoreType.DMA,
            pltpu.SemaphoreType.DMA,
        ],
        compiler_params=pltpu.CompilerParams(collective_id=0),
    )(x, w_mat)


# device time: 28092 ns/iter; 1.1205x vs baseline; 1.1205x over previous
import jax
import jax.numpy as jnp
from jax import lax
from jax.experimental import pallas as pl
from jax.experimental.pallas import tpu as pltpu

N_DEV = 32
W_BLOCKS = 8


def kernel(x, w_mat):
    m_per, kdim = x.shape
    n = w_mat.shape[1]
    n_per = n // N_DEV
    m_total = m_per * N_DEV
    wblk = n // W_BLOCKS
    t_per_b = wblk // n_per

    def body(
        x_ref, w_hbm, out_ref, wbuf, y_ref, wsems, send_sems, recv_sems,
        loc_sem,
    ):
        my = lax.axis_index("i")
        myq = lax.rem(my, t_per_b)
        rot = lax.div(my, t_per_b)

        barrier = pltpu.get_barrier_semaphore()
        for p in range(N_DEV):
            pl.semaphore_signal(
                barrier, inc=1,
                device_id=(p,), device_id_type=pl.DeviceIdType.MESH,
            )

        def w_copy(i):
            cb = lax.rem(rot + i, W_BLOCKS)
            return pltpu.make_async_copy(
                w_hbm.at[:, pl.ds(pl.multiple_of(cb * wblk, wblk), wblk)],
                wbuf.at[i % 2],
                wsems.at[i % 2],
            )

        w_copy(0).start()

        for i in range(W_BLOCKS):
            cb = lax.rem(rot + i, W_BLOCKS)
            if i + 1 < W_BLOCKS:
                w_copy(i + 1).start()
            w_copy(i).wait()

            yb = jnp.dot(
                x_ref[...], wbuf[i % 2], preferred_element_type=jnp.float32
            )
            yb = yb * jax.nn.sigmoid(yb)
            for j in range(t_per_b):
                y_ref[t_per_b * i + j] = yb[:, j * n_per:(j + 1) * n_per]

            if i == 0:
                pl.semaphore_wait(barrier, N_DEV)

            for j in range(t_per_b):
                k = t_per_b * i + j
                dest = t_per_b * cb + j

                @pl.when(dest != my)
                def _(k=k, dest=dest):
                    rdma = pltpu.make_async_remote_copy(
                        src_ref=y_ref.at[k],
                        dst_ref=out_ref.at[pl.ds(my * m_per, m_per), :],
                        send_sem=send_sems.at[k],
                        recv_sem=recv_sems.at[k],
                        device_id=(dest,),
                        device_id_type=pl.DeviceIdType.MESH,
                    )
                    rdma.start()

                @pl.when(dest == my)
                def _(k=k):
                    cp = pltpu.make_async_copy(
                        y_ref.at[k],
                        out_ref.at[pl.ds(my * m_per, m_per), :],
                        loc_sem,
                    )
                    cp.start()
                    cp.wait()

        for j in range(t_per_b):
            @pl.when(j != myq)
            def _(j=j):
                pltpu.make_async_copy(
                    y_ref.at[j], y_ref.at[j], send_sems.at[j]
                ).wait()
        for k in range(t_per_b, N_DEV):
            pltpu.make_async_copy(
                y_ref.at[0], y_ref.at[0], send_sems.at[k]
            ).wait()

        for i in range(W_BLOCKS):
            n_in = t_per_b - 1 if i == 0 else t_per_b
            for _ in range(n_in):
                recv = pltpu.make_async_remote_copy(
                    src_ref=y_ref.at[0],
                    dst_ref=out_ref.at[pl.ds(0, m_per), :],
                    send_sem=send_sems.at[0],
                    recv_sem=recv_sems.at[t_per_b * i + myq],
                    device_id=(0,),
                    device_id_type=pl.DeviceIdType.MESH,
                )
                recv.wait_recv()

    return pl.pallas_call(
        body,
        out_shape=jax.ShapeDtypeStruct((m_total, n_per), jnp.float32),
        in_specs=[
            pl.BlockSpec(memory_space=pltpu.VMEM),
            pl.BlockSpec(memory_space=pl.ANY),
        ],
        out_specs=pl.BlockSpec(memory_space=pl.ANY),
        scratch_shapes=[
            pltpu.VMEM((2, kdim, wblk), jnp.float32),
            pltpu.VMEM((N_DEV, m_per, n_per), jnp.float32),
            pltpu.SemaphoreType.DMA((2,)),
            pltpu.SemaphoreType.DMA((N_DEV,)),
            pltpu.SemaphoreType.DMA((N_DEV,)),
            pltpu.SemaphoreType.DMA,
        ],
        compiler_params=pltpu.CompilerParams(collective_id=0),
    )(x, w_mat)
